# baseline (speedup 1.0000x reference)
import functools

import jax
import jax.numpy as jnp
from jax import lax
from jax.experimental import pallas as pl
from jax.experimental.pallas import tpu as pltpu
from jax.experimental.pallas import tpu_sc as plsc


def _splat(tvec, idx):
    return lax.gather(
        tvec,
        jnp.full((16, 1), idx, jnp.int32),
        lax.GatherDimensionNumbers(
            offset_dims=(), collapsed_slice_dims=(0,), start_index_map=(0,)
        ),
        (1,),
        mode=lax.GatherScatterMode.PROMISE_IN_BOUNDS,
    )


def _make_sc_add(L, D, B):
    NC = 2
    NW = 32
    OCT = D // 8
    UNITS = L * OCT
    UPW = UNITS // NW
    mesh = plsc.VectorSubcoreMesh(core_axis_name="c", subcore_axis_name="s")

    @functools.partial(
        pl.kernel,
        mesh=mesh,
        out_type=jax.ShapeDtypeStruct((L, D, B), jnp.float32),
        scratch_types=[
            pltpu.VMEM((2, 8, B), jnp.float32),
            pltpu.VMEM((16,), jnp.float32),
            pltpu.SemaphoreType.DMA,
            pltpu.SemaphoreType.DMA,
        ],
    )
    def sc_add(x_hbm, t_hbm, out_hbm, buf, tv, isem, osem):
        wid = lax.axis_index("s") * NC + lax.axis_index("c")

        def unit(j, _):
            u = wid * UPW + j
            l = u // OCT
            a = u % OCT
            pltpu.sync_copy(
                t_hbm.at[pl.ds(l * D + 8 * a, 8)], tv.at[pl.ds(0, 8)]
            )
            tvec = tv[...]
            pltpu.async_copy(
                x_hbm.at[l, pl.ds(8 * a, 8), :], buf.at[0], isem
            ).wait()
            vals = [_splat(tvec, r) for r in range(8)]

            def col(c, _):
                base = c * 16
                for r in range(8):
                    sl = pl.ds(base, 16)
                    buf[0, r, sl] = buf[0, r, sl] + vals[r]
                return _

            lax.fori_loop(0, B // 16, col, 0)
            pltpu.async_copy(
                buf.at[0], out_hbm.at[l, pl.ds(8 * a, 8), :], osem
            ).wait()
            return _

        lax.fori_loop(0, UPW, unit, 0)

    return sc_add


def kernel(x, pos_table):
    B, L, D = x.shape
    xt = x.transpose(1, 2, 0)
    tflat = pos_table.reshape(-1)
    out_t = _make_sc_add(L, D, B)(xt, tflat)
    return out_t.transpose(2, 0, 1)


# speedup vs baseline: 2.8298x; 2.8298x over previous
import jax
import jax.numpy as jnp
from jax.experimental import pallas as pl
from jax.experimental.pallas import tpu as pltpu


_LCHUNK = 8


def _add_body(x_ref, t_ref, o_ref):
    o_ref[...] = x_ref[...] + jax.lax.broadcast_in_dim(
        t_ref[...], o_ref.shape, (0, 1)
    )


def kernel(x, pos_table):
    B, L, D = x.shape
    xt = x.transpose(1, 2, 0)
    out_t = pl.pallas_call(
        _add_body,
        grid=(L // _LCHUNK,),
        in_specs=[
            pl.BlockSpec((_LCHUNK, D, B), lambda i: (i, 0, 0)),
            pl.BlockSpec((_LCHUNK, D), lambda i: (i, 0)),
        ],
        out_specs=pl.BlockSpec((_LCHUNK, D, B), lambda i: (i, 0, 0)),
        out_shape=jax.ShapeDtypeStruct((L, D, B), x.dtype),
        compiler_params=pltpu.CompilerParams(
            dimension_semantics=("arbitrary",),
        ),
    )(xt, pos_table)
    return out_t.transpose(2, 0, 1)
